# baseline (speedup 1.0000x reference)
import functools

import jax
import jax.numpy as jnp
from jax.experimental import pallas as pl
from jax.experimental.pallas import tpu as pltpu

BM = 512


def _encode_body(x_ref, w_ref, b_ref, o_ref):
    xb = x_ref[...].astype(jnp.bfloat16)
    for v in range(3):
        s = jnp.dot(xb, w_ref[v], preferred_element_type=jnp.float32)
        o_ref[v] = jax.nn.relu(s + b_ref[v:v + 1, :])


def _intra_body(a_ref, h_ref, wz0, wz1, wr0, wr1, wh0, wh1, b_ref, m_ref,
                o_ref):
    i = pl.program_id(0)
    h_full = h_ref[...]
    hb = h_full.astype(jnp.bfloat16)
    a = jnp.dot(a_ref[...].astype(jnp.bfloat16), hb,
                preferred_element_type=jnp.float32)
    ab = a.astype(jnp.bfloat16)
    hloc = h_ref[pl.ds(i * BM, BM), :]
    hlb = hloc.astype(jnp.bfloat16)
    z = jax.nn.sigmoid(
        jnp.dot(ab, wz0[...], preferred_element_type=jnp.float32)
        + jnp.dot(hlb, wz1[...], preferred_element_type=jnp.float32)
        + b_ref[0:1, :])
    r = jax.nn.sigmoid(
        jnp.dot(ab, wr0[...], preferred_element_type=jnp.float32)
        + jnp.dot(hlb, wr1[...], preferred_element_type=jnp.float32)
        + b_ref[1:2, :])
    hh = jax.nn.relu(m_ref[...] * (
        jnp.dot(ab, wh0[...], preferred_element_type=jnp.float32)
        + jnp.dot((r * hloc).astype(jnp.bfloat16), wh1[...],
                  preferred_element_type=jnp.float32)
        + b_ref[2:3, :]))
    o_ref[...] = hh * z + hloc * (1.0 - z)


def _final_body(s_ref, wt_ref, bt_ref, wa_ref, ba_ref, o_ref):
    t = []
    for v in range(3):
        sb = s_ref[v].astype(jnp.bfloat16)
        t.append(jnp.dot(sb, wt_ref[v], preferred_element_type=jnp.float32)
                 + bt_ref[v:v + 1, :])
    acc = None
    for v in range(3):
        av = t[v] + t[(v + 1) % 3]
        av = jnp.where(av >= 0, av, 0.2 * av)
        o = jnp.dot(av.astype(jnp.bfloat16), wa_ref[v],
                    preferred_element_type=jnp.float32) + ba_ref[v:v + 1, :]
        acc = o if acc is None else acc + o
    o_ref[...] = acc


def _intra_call(A, h, wz0, wz1, wr0, wr1, wh0, wh1, b, mask):
    n, d = h.shape
    nb = n // BM
    wspec = pl.BlockSpec((d, d), lambda i: (0, 0))
    return pl.pallas_call(
        _intra_body,
        grid=(nb,),
        in_specs=[
            pl.BlockSpec((BM, n), lambda i: (i, 0)),
            pl.BlockSpec((n, d), lambda i: (0, 0)),
            wspec, wspec, wspec, wspec, wspec, wspec,
            pl.BlockSpec((3, d), lambda i: (0, 0)),
            pl.BlockSpec((BM, 1), lambda i: (i, 0)),
        ],
        out_specs=pl.BlockSpec((BM, d), lambda i: (i, 0)),
        out_shape=jax.ShapeDtypeStruct((n, d), jnp.float32),
        compiler_params=pltpu.CompilerParams(
            dimension_semantics=("arbitrary",)),
    )(A, h, wz0, wz1, wr0, wr1, wh0, wh1, b, mask)


def kernel(x, adj, adj1, adj2, mask, mask1, mask2, params):
    n, d = x.shape[1], x.shape[2]
    x2 = x[0]
    adjs = (adj[0], adj1[0], adj2[0])
    masks = (mask[0], mask1[0], mask2[0])
    p = params

    we = jnp.stack([p['weights_encode_%d' % v] for v in range(3)]
                   ).astype(jnp.bfloat16)
    be = jnp.stack([p['bias_encode_%d' % v] for v in range(3)])
    s0 = pl.pallas_call(
        _encode_body,
        out_shape=jax.ShapeDtypeStruct((3, n, d), jnp.float32),
    )(x2, we, be)

    wg = {}
    bg = {}
    for v in range(3):
        wg[v] = [p['weights_%d_%s' % (v, suf)].astype(jnp.bfloat16)
                 for suf in ('z0', 'z1', 'r0', 'r1', 'h0', 'h1')]
        bg[v] = jnp.stack([
            p['bias_%d_z0' % v] + p['bias_%d_z1' % v],
            p['bias_%d_r0' % v] + p['bias_%d_r1' % v],
            p['bias_%d_h0' % v] + p['bias_%d_h1' % v],
        ])

    hs = [s0[v] for v in range(3)]
    for _ in range(2):
        hs = [_intra_call(adjs[v], hs[v], *wg[v], bg[v], masks[v])
              for v in range(3)]

    wt = jnp.stack([p['weights_%d%d' % (v, v)] for v in range(3)]
                   ).astype(jnp.bfloat16)
    bt = jnp.stack([p['bias_%d%d' % (v, v)] for v in range(3)])
    wa = jnp.stack([p['weights_att%d' % v] for v in range(3)]
                   ).astype(jnp.bfloat16)
    ba = jnp.stack([p['bias_att%d' % v] for v in range(3)])
    out = pl.pallas_call(
        _final_body,
        out_shape=jax.ShapeDtypeStruct((n, d), jnp.float32),
    )(jnp.stack(hs), wt, bt, wa, ba)
    return out[None]


# speedup vs baseline: 1.0621x; 1.0621x over previous
import functools

import jax
import jax.numpy as jnp
from jax.experimental import pallas as pl
from jax.experimental.pallas import tpu as pltpu

BM = 512


def _encode_body(x_ref, w_ref, b_ref, o_ref):
    x = x_ref[...]
    for v in range(3):
        s = jnp.dot(x, w_ref[v], preferred_element_type=jnp.float32)
        o_ref[v] = jax.nn.relu(s + b_ref[v:v + 1, :])


def _intra_body(a_ref, h_ref, wz0, wz1, wr0, wr1, wh0, wh1, b_ref, m_ref,
                o_ref):
    i = pl.program_id(0)
    h_full = h_ref[...]
    a = jnp.dot(a_ref[...], h_full, preferred_element_type=jnp.float32)
    hloc = h_ref[pl.ds(i * BM, BM), :]
    z = jax.nn.sigmoid(
        jnp.dot(a, wz0[...], preferred_element_type=jnp.float32)
        + jnp.dot(hloc, wz1[...], preferred_element_type=jnp.float32)
        + b_ref[0:1, :])
    r = jax.nn.sigmoid(
        jnp.dot(a, wr0[...], preferred_element_type=jnp.float32)
        + jnp.dot(hloc, wr1[...], preferred_element_type=jnp.float32)
        + b_ref[1:2, :])
    hh = jax.nn.relu(m_ref[...] * (
        jnp.dot(a, wh0[...], preferred_element_type=jnp.float32)
        + jnp.dot(r * hloc, wh1[...], preferred_element_type=jnp.float32)
        + b_ref[2:3, :]))
    o_ref[...] = hh * z + hloc * (1.0 - z)


def _final_body(s_ref, wt_ref, bt_ref, wa_ref, ba_ref, o_ref):
    t = []
    for v in range(3):
        t.append(jnp.dot(s_ref[v], wt_ref[v],
                         preferred_element_type=jnp.float32)
                 + bt_ref[v:v + 1, :])
    acc = None
    for v in range(3):
        av = t[v] + t[(v + 1) % 3]
        av = jnp.where(av >= 0, av, 0.2 * av)
        o = jnp.dot(av, wa_ref[v],
                    preferred_element_type=jnp.float32) + ba_ref[v:v + 1, :]
        acc = o if acc is None else acc + o
    o_ref[...] = acc


def _intra_call(A, h, wz0, wz1, wr0, wr1, wh0, wh1, b, mask):
    n, d = h.shape
    nb = n // BM
    wspec = pl.BlockSpec((d, d), lambda i: (0, 0))
    return pl.pallas_call(
        _intra_body,
        grid=(nb,),
        in_specs=[
            pl.BlockSpec((BM, n), lambda i: (i, 0)),
            pl.BlockSpec((n, d), lambda i: (0, 0)),
            wspec, wspec, wspec, wspec, wspec, wspec,
            pl.BlockSpec((3, d), lambda i: (0, 0)),
            pl.BlockSpec((BM, 1), lambda i: (i, 0)),
        ],
        out_specs=pl.BlockSpec((BM, d), lambda i: (i, 0)),
        out_shape=jax.ShapeDtypeStruct((n, d), jnp.float32),
        compiler_params=pltpu.CompilerParams(
            dimension_semantics=("arbitrary",)),
    )(A, h, wz0, wz1, wr0, wr1, wh0, wh1, b, mask)


def kernel(x, adj, adj1, adj2, mask, mask1, mask2, params):
    n, d = x.shape[1], x.shape[2]
    x2 = x[0]
    adjs = (adj[0], adj1[0], adj2[0])
    masks = (mask[0], mask1[0], mask2[0])
    p = params

    we = jnp.stack([p['weights_encode_%d' % v] for v in range(3)])
    be = jnp.stack([p['bias_encode_%d' % v] for v in range(3)])
    s0 = pl.pallas_call(
        _encode_body,
        out_shape=jax.ShapeDtypeStruct((3, n, d), jnp.float32),
    )(x2, we, be)

    wg = {}
    bg = {}
    for v in range(3):
        wg[v] = [p['weights_%d_%s' % (v, suf)]
                 for suf in ('z0', 'z1', 'r0', 'r1', 'h0', 'h1')]
        bg[v] = jnp.stack([
            p['bias_%d_z0' % v] + p['bias_%d_z1' % v],
            p['bias_%d_r0' % v] + p['bias_%d_r1' % v],
            p['bias_%d_h0' % v] + p['bias_%d_h1' % v],
        ])

    hs = [s0[v] for v in range(3)]
    for _ in range(2):
        hs = [_intra_call(adjs[v], hs[v], *wg[v], bg[v], masks[v])
              for v in range(3)]

    wt = jnp.stack([p['weights_%d%d' % (v, v)] for v in range(3)])
    bt = jnp.stack([p['bias_%d%d' % (v, v)] for v in range(3)])
    wa = jnp.stack([p['weights_att%d' % v] for v in range(3)])
    ba = jnp.stack([p['bias_att%d' % v] for v in range(3)])
    out = pl.pallas_call(
        _final_body,
        out_shape=jax.ShapeDtypeStruct((n, d), jnp.float32),
    )(jnp.stack(hs), wt, bt, wa, ba)
    return out[None]


# speedup vs baseline: 1.2390x; 1.1665x over previous
import functools

import jax
import jax.numpy as jnp
from jax.experimental import pallas as pl
from jax.experimental.pallas import tpu as pltpu

BM = 512


def _encode_body(x_ref, w_ref, b_ref, o_ref):
    x = x_ref[...]
    for v in range(3):
        s = jnp.dot(x, w_ref[v], preferred_element_type=jnp.float32)
        o_ref[v] = jax.nn.relu(s + b_ref[v:v + 1, :])


def _gru(a, hloc, wz0, wz1, wr0, wr1, wh0, wh1, b_ref, m):
    z = jax.nn.sigmoid(
        jnp.dot(a, wz0[...], preferred_element_type=jnp.float32)
        + jnp.dot(hloc, wz1[...], preferred_element_type=jnp.float32)
        + b_ref[0:1, :])
    r = jax.nn.sigmoid(
        jnp.dot(a, wr0[...], preferred_element_type=jnp.float32)
        + jnp.dot(hloc, wr1[...], preferred_element_type=jnp.float32)
        + b_ref[1:2, :])
    hh = jax.nn.relu(m * (
        jnp.dot(a, wh0[...], preferred_element_type=jnp.float32)
        + jnp.dot(r * hloc, wh1[...], preferred_element_type=jnp.float32)
        + b_ref[2:3, :]))
    return hh * z + hloc * (1.0 - z)


def _steps_body(a_ref, h_ref, wz0, wz1, wr0, wr1, wh0, wh1, b_ref, m_ref,
                o_ref, sa_ref, sh_ref):
    s = pl.program_id(0)
    i = pl.program_id(1)
    gates = (wz0, wz1, wr0, wr1, wh0, wh1, b_ref, m_ref[...])

    @pl.when(s == 0)
    def _pass1():
        blk = a_ref[...]
        sa_ref[pl.ds(i * BM, BM), :] = blk.astype(jnp.bfloat16)
        a = jnp.dot(blk, h_ref[...], preferred_element_type=jnp.float32)
        hloc = h_ref[pl.ds(i * BM, BM), :]
        h2 = _gru(a, hloc, *gates)
        sh_ref[pl.ds(i * BM, BM), :] = h2
        o_ref[...] = h2

    @pl.when(s == 1)
    def _pass2():
        h2b = sh_ref[...].astype(jnp.bfloat16)
        a = jnp.dot(sa_ref[pl.ds(i * BM, BM), :], h2b,
                    preferred_element_type=jnp.float32)
        hloc = sh_ref[pl.ds(i * BM, BM), :]
        o_ref[...] = _gru(a, hloc, *gates)


def _final_body(s_ref, wt_ref, bt_ref, wa_ref, ba_ref, o_ref):
    t = []
    for v in range(3):
        t.append(jnp.dot(s_ref[v], wt_ref[v],
                         preferred_element_type=jnp.float32)
                 + bt_ref[v:v + 1, :])
    acc = None
    for v in range(3):
        av = t[v] + t[(v + 1) % 3]
        av = jnp.where(av >= 0, av, 0.2 * av)
        o = jnp.dot(av, wa_ref[v],
                    preferred_element_type=jnp.float32) + ba_ref[v:v + 1, :]
        acc = o if acc is None else acc + o
    o_ref[...] = acc


def _steps_call(A, h, wz0, wz1, wr0, wr1, wh0, wh1, b, mask):
    n, d = h.shape
    nb = n // BM
    wspec = pl.BlockSpec((d, d), lambda s, i: (0, 0))
    return pl.pallas_call(
        _steps_body,
        grid=(2, nb),
        in_specs=[
            pl.BlockSpec((BM, n),
                         lambda s, i: (jnp.where(s == 0, i, nb - 1), 0)),
            pl.BlockSpec((n, d), lambda s, i: (0, 0)),
            wspec, wspec, wspec, wspec, wspec, wspec,
            pl.BlockSpec((3, d), lambda s, i: (0, 0)),
            pl.BlockSpec((BM, 1), lambda s, i: (i, 0)),
        ],
        out_specs=pl.BlockSpec((BM, d), lambda s, i: (i, 0)),
        out_shape=jax.ShapeDtypeStruct((n, d), jnp.float32),
        scratch_shapes=[
            pltpu.VMEM((n, n), jnp.bfloat16),
            pltpu.VMEM((n, d), jnp.float32),
        ],
        compiler_params=pltpu.CompilerParams(
            dimension_semantics=("arbitrary", "arbitrary")),
    )(A, h, wz0, wz1, wr0, wr1, wh0, wh1, b, mask)


def kernel(x, adj, adj1, adj2, mask, mask1, mask2, params):
    n, d = x.shape[1], x.shape[2]
    x2 = x[0]
    adjs = (adj[0], adj1[0], adj2[0])
    masks = (mask[0], mask1[0], mask2[0])
    p = params

    we = jnp.stack([p['weights_encode_%d' % v] for v in range(3)])
    be = jnp.stack([p['bias_encode_%d' % v] for v in range(3)])
    s0 = pl.pallas_call(
        _encode_body,
        out_shape=jax.ShapeDtypeStruct((3, n, d), jnp.float32),
    )(x2, we, be)

    hs = []
    for v in range(3):
        wg = [p['weights_%d_%s' % (v, suf)]
              for suf in ('z0', 'z1', 'r0', 'r1', 'h0', 'h1')]
        bg = jnp.stack([
            p['bias_%d_z0' % v] + p['bias_%d_z1' % v],
            p['bias_%d_r0' % v] + p['bias_%d_r1' % v],
            p['bias_%d_h0' % v] + p['bias_%d_h1' % v],
        ])
        hs.append(_steps_call(adjs[v], s0[v], *wg, bg, masks[v]))

    wt = jnp.stack([p['weights_%d%d' % (v, v)] for v in range(3)])
    bt = jnp.stack([p['bias_%d%d' % (v, v)] for v in range(3)])
    wa = jnp.stack([p['weights_att%d' % v] for v in range(3)])
    ba = jnp.stack([p['bias_att%d' % v] for v in range(3)])
    out = pl.pallas_call(
        _final_body,
        out_shape=jax.ShapeDtypeStruct((n, d), jnp.float32),
    )(jnp.stack(hs), wt, bt, wa, ba)
    return out[None]
